# Initial kernel scaffold; baseline (speedup 1.0000x reference)
import functools
import jax
import jax.numpy as jnp
from jax import lax
from jax.experimental import pallas as pl
from jax.experimental.pallas import tpu as pltpu, tpu_sc as plsc


def kernel(atom_description, coords, alternatives, mean, std, weight):
    mesh = plsc.VectorSubcoreMesh(core_axis_name="c", subcore_axis_name="s")

    @functools.partial(
        pl.kernel, mesh=mesh,
        out_type=jax.ShapeDtypeStruct((32, 16), jnp.int32),
        scratch_types=[pltpu.VMEM((16,), jnp.int32)],
    )
    def sck(out_hbm, buf):
        wid = lax.axis_index("s") * 2 + lax.axis_index("c")
        buf[...] = jnp.full((16,), -1, jnp.int32)
        lane = lax.iota(jnp.int32, 16)
        dest = lane // 4
        plsc.store_scatter(buf, [dest], lane)
        dest2 = jnp.full((16,), 8, jnp.int32)
        plsc.store_scatter(buf, [dest2], lane + 100)
        pltpu.sync_copy(buf, out_hbm.at[wid])

    res = sck()
    out = jnp.zeros((8, 4, 8192, 2), jnp.float32)
    return out + res[0, 0].astype(jnp.float32)


# speedup vs baseline: 1.3434x; 1.3434x over previous
import functools
import jax
import jax.numpy as jnp
from jax import lax
from jax.experimental import pallas as pl
from jax.experimental.pallas import tpu as pltpu

PAD = 999.0
PAD_I = 999
EPS = 1e-10
NRES = 20
NB, NC, NR = 8, 4, 8192
BC = NB * NC

_A0, _A1, _A2, _A3 = 1.5707288, -0.2121144, 0.0742610, -0.0187293


def _acos(x):
    ax = jnp.abs(x)
    p = ((_A3 * ax + _A2) * ax + _A1) * ax + _A0
    r = jnp.sqrt(jnp.maximum(1.0 - ax, 0.0)) * p
    return jnp.where(x >= 0, r, jnp.float32(3.14159265358979) - r)


def _geom_body(g_ref, tab_ref, out_ref):
    g = g_ref[0, 0]
    nx, ny, nz = g[0:1], g[1:2], g[2:3]
    cpx, cpy, cpz = g[3:4], g[4:5], g[5:6]
    cax, cay, caz = g[6:7], g[7:8], g[8:9]
    capx, capy, capz = g[9:10], g[10:11], g[11:12]
    sc, sp = g[12:13], g[13:14]

    todo = (nx != PAD) & (cpx != PAD) & (cax != PAD) & (capx != PAD) \
        & (sc != jnp.float32(PAD_I)) & (sp != jnp.float32(PAD_I))

    v1x, v1y, v1z = cpx - nx, cpy - ny, cpz - nz
    v2x, v2y, v2z = cax - nx, cay - ny, caz - nz
    w1x, w1y, w1z = capx - cpx, capy - cpy, capz - cpz

    n1sq = v1x * v1x + v1y * v1y + v1z * v1z
    bond = jnp.sqrt(n1sq)

    d1 = v1x * v2x + v1y * v2y + v1z * v2z
    n2 = jnp.sqrt(v2x * v2x + v2y * v2y + v2z * v2z)
    nrm1 = bond * n2
    cos1 = jnp.clip(d1 / jnp.clip(nrm1, 1e-8, None), -0.9999999, 0.9999999)
    a1 = _acos(cos1)

    d2 = -(w1x * v1x + w1y * v1y + w1z * v1z)
    n3 = jnp.sqrt(w1x * w1x + w1y * w1y + w1z * w1z)
    nrm2 = n3 * bond
    cos2 = jnp.clip(d2 / jnp.clip(nrm2, 1e-8, None), -0.9999999, 0.9999999)
    a2 = _acos(cos2)

    seq_sel = jnp.where(todo, sc, 0.0).astype(jnp.int32)
    row = lax.broadcasted_iota(jnp.int32, (32, 8192), 0)
    oh = jnp.where(row == seq_sel, 1.0, 0.0)
    gathered = jnp.dot(tab_ref[...], oh,
                       preferred_element_type=jnp.float32)

    total = jnp.zeros((1, 8192), jnp.float32)
    xs = (bond, a1, a2)
    for i in range(3):
        m = gathered[i:i + 1]
        iv = gathered[3 + i:4 + i]
        tcap = gathered[6 + i:7 + i]
        d = xs[i] - m
        total = total + jnp.minimum(d * d * iv, tcap)

    out_ref[0, 0] = jnp.where(todo, total, 0.0)


def _geom(gf, tab):
    return pl.pallas_call(
        _geom_body,
        grid=(2, BC),
        in_specs=[
            pl.BlockSpec((1, 1, 14, NR), lambda a, b: (a, b, 0, 0)),
            pl.BlockSpec((9, 32), lambda a, b: (0, 0)),
        ],
        out_specs=pl.BlockSpec((1, 1, 1, NR), lambda a, b: (a, b, 0, 0)),
        out_shape=jax.ShapeDtypeStruct((2, BC, 1, NR), jnp.float32),
    )(gf, tab)


def _masks_from(atname, mask):
    cls = atname % 4
    unmasked = atname < 3
    ok = mask | unmasked
    nm = (cls == 0) & ok
    cm = (cls == 1) & ok
    cam = (cls == 2) & ok
    return nm, cm, cam


def kernel(atom_description, coords, alternatives, mean, std, weight):
    resnum = atom_description[:, 0]
    atname = atom_description[:, 1]
    chain = atom_description[:, 2]
    resname = atom_description[:, 3]
    batch_ind = atom_description[:, 4]

    factor = 1.0 - jnp.tanh(-weight[0])
    var = std.astype(jnp.float32) ** 2
    iv = factor / (2.0 * var)
    tcap = factor * (-jnp.log(jnp.float32(EPS))
                     - jnp.log(jnp.sqrt(2.0 * jnp.pi * var)))
    tab = jnp.zeros((9, 32), jnp.float32)
    tab = tab.at[0:3, :20].set(mean.astype(jnp.float32).T)
    tab = tab.at[3:6, :20].set(iv.T)
    tab = tab.at[6:9, :20].set(tcap.T)

    chans = []
    for alt in range(2):
        mask = alternatives[..., alt]
        nm, cm, cam = _masks_from(atname, mask)
        b_nm = jnp.where(nm, batch_ind, NB)
        b_cm = jnp.where(cm, batch_ind, NB)
        b_cam = jnp.where(cam, batch_ind, NB)
        Narray = jnp.full((NB, NC, NR, 3), PAD, jnp.float32)
        Carray = jnp.full((NB, NC, NR, 3), PAD, jnp.float32)
        CAarray = jnp.full((NB, NC, NR, 3), PAD, jnp.float32)
        seq = jnp.full((NB, NC, NR), PAD_I, jnp.int32)
        Narray = Narray.at[b_nm, chain, resnum].set(coords, mode='drop')
        Carray = Carray.at[b_cm, chain, resnum].set(coords, mode='drop')
        CAarray = CAarray.at[b_cam, chain, resnum].set(coords, mode='drop')
        seq = seq.at[b_cam, chain, resnum].set(
            resname.astype(jnp.int32), mode='drop')

        N2 = Narray.reshape(BC, NR, 3)
        C2 = Carray.reshape(BC, NR, 3)
        CA2 = CAarray.reshape(BC, NR, 3)
        sqf = seq.reshape(BC, NR).astype(jnp.float32)

        padc = jnp.full((BC, 1), PAD, jnp.float32)
        padi = jnp.full((BC, 1), float(PAD_I), jnp.float32)

        ch = [N2[:, :, 0], N2[:, :, 1], N2[:, :, 2]]
        for k in range(3):
            ch.append(jnp.concatenate([padc, C2[:, :-1, k]], axis=1))
        ch += [CA2[:, :, 0], CA2[:, :, 1], CA2[:, :, 2]]
        for k in range(3):
            ch.append(jnp.concatenate([padc, CA2[:, :-1, k]], axis=1))
        ch.append(sqf)
        ch.append(jnp.concatenate([padi, sqf[:, :-1]], axis=1))
        chans.append(jnp.stack(ch, axis=1))

    gf = jnp.stack(chans, axis=0)
    out = _geom(gf, tab)
    out = out.reshape(2, NB, NC, NR).transpose(1, 2, 3, 0)
    return out


# speedup vs baseline: 85.0711x; 63.3250x over previous
import functools
import jax
import jax.numpy as jnp
from jax import lax
from jax.experimental import pallas as pl
from jax.experimental.pallas import tpu as pltpu, tpu_sc as plsc

PAD = 999.0
PAD_I = 999
EPS = 1e-10
NB, NC, NR = 8, 4, 8192
BC = NB * NC
NA = 2097152
TILE_SLOTS = 3 * NR
CH = 8192
NCHUNK = NA // CH
VPC = CH // 16
FCAP = 1072

_A0, _A1, _A2, _A3 = 1.5707288, -0.2121144, 0.0742610, -0.0187293


def _pack_body(rn_ref, an_ref, chn_ref, bt_ref, a0_ref, a1_ref, key_ref):
    rn = rn_ref[...]
    an = an_ref[...]
    chn = chn_ref[...]
    bt = bt_ref[...]
    a0 = a0_ref[...]
    a1 = a1_ref[...]
    cls = an & 3
    validc = cls < 3
    unmasked = an < 3
    v0 = validc & ((a0 != 0) | unmasked)
    v1 = validc & ((a1 != 0) | unmasked)
    dest = (bt * NC + chn) * TILE_SLOTS + cls * NR + rn
    key = dest | (v0.astype(jnp.int32) << 30) | (v1.astype(jnp.int32) << 31)
    key_ref[...] = key


def _pack_keys(rn, an, chn, bt, a0, a1):
    spec = pl.BlockSpec((32, NR), lambda i: (i, 0))
    return pl.pallas_call(
        _pack_body,
        grid=(NA // (32 * NR),),
        in_specs=[spec] * 6,
        out_specs=spec,
        out_shape=jax.ShapeDtypeStruct((NA // NR, NR), jnp.int32),
    )(rn, an, chn, bt, a0, a1)


def _sc_body(key_hbm, cx_hbm, cy_hbm, cz_hbm, rn_hbm, gf_hbm,
             w0, w1, kbuf, fk, fi, idxb, vx, vy, vz, si, sem0, sem1, gsem):
    wid = lax.axis_index("s") * 2 + lax.axis_index("c")
    base = wid * TILE_SLOTS
    lane = lax.iota(jnp.int32, 16)
    neg1 = jnp.full((16,), -1, jnp.int32)
    sems = [sem0, sem1]

    def initb(i, c):
        w0[pl.ds(i * 16, 16)] = neg1
        w1[pl.ds(i * 16, 16)] = neg1
        return c
    lax.fori_loop(0, TILE_SLOTS // 16, initb, 0)

    zero16 = jnp.zeros((16,), jnp.int32)

    def initf(i, c):
        fk[pl.ds(i * 16, 16)] = zero16
        fi[pl.ds(i * 16, 16)] = zero16
        return c
    lax.fori_loop(0, FCAP // 16, initf, 0)

    pltpu.make_async_copy(key_hbm.at[pl.ds(0, CH)], kbuf.at[0], sem0).start()

    def chunk_body(g2, c0):
        for ph in range(2):
            g = g2 * 2 + ph
            nxt = g + 1

            @pl.when(nxt < NCHUNK)
            def _():
                pltpu.make_async_copy(
                    key_hbm.at[pl.ds(nxt * CH, CH)],
                    kbuf.at[1 - ph], sems[1 - ph]).start()

            pltpu.make_async_copy(
                key_hbm.at[pl.ds(g * CH, CH)], kbuf.at[ph], sems[ph]).wait()
            gbase = g * CH

            def vbody(j, iv):
                k = kbuf[ph, pl.ds(j * 16, 16)]
                local = (k & 0xFFFFF) - base
                m_in = local.astype(jnp.uint32) < jnp.uint32(TILE_SLOTS)
                lc = jnp.minimum(jnp.maximum(local, 0),
                                 jnp.int32(TILE_SLOTS - 1))
                m0 = m_in & ((k & (1 << 30)) != 0)
                m1 = m_in & (k < 0)
                plsc.store_scatter(w0, [lc], iv, mask=m0)
                plsc.store_scatter(w1, [lc], iv, mask=m1)
                return iv + 16
            lax.fori_loop(0, VPC, vbody, gbase + lane, unroll=4)
        return c0
    lax.fori_loop(0, NCHUNK // 2, chunk_body, 0)

    ws = [w0, w1]
    for a in range(2):
        for cls in range(3):
            woff = cls * NR

            def fixb(j, c, _a=a, _woff=woff):
                w = ws[_a][pl.ds(_woff + j * 16, 16)]
                safe = jnp.where(w < 0, j * 16 + lane, w)
                idxb[pl.ds(j * 16, 16)] = safe
                return c
            lax.fori_loop(0, NR // 16, fixb, 0)

            pltpu.make_async_copy(cx_hbm.at[idxb], vx, gsem).start()
            pltpu.make_async_copy(cy_hbm.at[idxb], vy, gsem).start()
            pltpu.make_async_copy(cz_hbm.at[idxb], vz, gsem).start()
            if cls == 2:
                pltpu.make_async_copy(rn_hbm.at[idxb], si, gsem).start()
            pltpu.make_async_copy(cx_hbm.at[idxb], vx, gsem).wait()
            pltpu.make_async_copy(cy_hbm.at[idxb], vy, gsem).wait()
            pltpu.make_async_copy(cz_hbm.at[idxb], vz, gsem).wait()
            if cls == 2:
                pltpu.make_async_copy(rn_hbm.at[idxb], si, gsem).wait()

            def comb(j, c, _a=a, _woff=woff):
                w = ws[_a][pl.ds(_woff + j * 16, 16)]
                emp = w < 0
                sl = pl.ds(j * 16, 16)
                vx[sl] = jnp.where(emp, PAD, vx[sl])
                vy[sl] = jnp.where(emp, PAD, vy[sl])
                vz[sl] = jnp.where(emp, PAD, vz[sl])
                return c
            lax.fori_loop(0, NR // 16, comb, 0)

            pltpu.sync_copy(vx, gf_hbm.at[a, wid, cls * 3 + 0])
            pltpu.sync_copy(vy, gf_hbm.at[a, wid, cls * 3 + 1])
            pltpu.sync_copy(vz, gf_hbm.at[a, wid, cls * 3 + 2])
            if cls == 2:
                def combs(j, c, _a=a, _woff=woff):
                    w = ws[_a][pl.ds(_woff + j * 16, 16)]
                    emp = w < 0
                    sl = pl.ds(j * 16, 16)
                    vx[sl] = jnp.where(emp, jnp.float32(PAD_I),
                                       si[sl].astype(jnp.float32))
                    return c
                lax.fori_loop(0, NR // 16, combs, 0)
                pltpu.sync_copy(vx, gf_hbm.at[a, wid, 9])


def _sc_build(keys, cx, cy, cz, rn):
    mesh = plsc.VectorSubcoreMesh(core_axis_name="c", subcore_axis_name="s")
    f = functools.partial(
        pl.kernel,
        out_type=jax.ShapeDtypeStruct((2, BC, 10, NR), jnp.float32),
        mesh=mesh,
        scratch_types=[
            pltpu.VMEM((TILE_SLOTS,), jnp.int32),
            pltpu.VMEM((TILE_SLOTS,), jnp.int32),
            pltpu.VMEM((2, CH), jnp.int32),
            pltpu.VMEM((FCAP,), jnp.int32),
            pltpu.VMEM((FCAP,), jnp.int32),
            pltpu.VMEM((NR,), jnp.int32),
            pltpu.VMEM((NR,), jnp.float32),
            pltpu.VMEM((NR,), jnp.float32),
            pltpu.VMEM((NR,), jnp.float32),
            pltpu.VMEM((NR,), jnp.int32),
            pltpu.SemaphoreType.DMA,
            pltpu.SemaphoreType.DMA,
            pltpu.SemaphoreType.DMA,
        ],
        compiler_params=pltpu.CompilerParams(needs_layout_passes=False),
    )(_sc_body)
    return f(keys, cx, cy, cz, rn)


def _acos(x):
    ax = jnp.abs(x)
    p = ((_A3 * ax + _A2) * ax + _A1) * ax + _A0
    r = jnp.sqrt(jnp.maximum(1.0 - ax, 0.0)) * p
    return jnp.where(x >= 0, r, jnp.float32(3.14159265358979) - r)


def _geom_body(g_ref, tab_ref, out_ref):
    g = g_ref[0, 0]
    nx, ny, nz = g[0:1], g[1:2], g[2:3]
    pad3 = jnp.full((3, 1), PAD, jnp.float32)
    cprev = jnp.concatenate([pad3, g[3:6, :NR - 1]], axis=1)
    cpx, cpy, cpz = cprev[0:1], cprev[1:2], cprev[2:3]
    cax, cay, caz = g[6:7], g[7:8], g[8:9]
    caprev = jnp.concatenate([pad3, g[6:9, :NR - 1]], axis=1)
    capx, capy, capz = caprev[0:1], caprev[1:2], caprev[2:3]
    sc = g[9:10]
    sp = jnp.concatenate([jnp.full((1, 1), float(PAD_I), jnp.float32),
                          sc[:, :NR - 1]], axis=1)

    todo = (nx != PAD) & (cpx != PAD) & (cax != PAD) & (capx != PAD) \
        & (sc != jnp.float32(PAD_I)) & (sp != jnp.float32(PAD_I))

    v1x, v1y, v1z = cpx - nx, cpy - ny, cpz - nz
    v2x, v2y, v2z = cax - nx, cay - ny, caz - nz
    w1x, w1y, w1z = capx - cpx, capy - cpy, capz - cpz

    bond = jnp.sqrt(v1x * v1x + v1y * v1y + v1z * v1z)

    d1 = v1x * v2x + v1y * v2y + v1z * v2z
    n2 = jnp.sqrt(v2x * v2x + v2y * v2y + v2z * v2z)
    cos1 = jnp.clip(d1 / jnp.clip(bond * n2, 1e-8, None),
                    -0.9999999, 0.9999999)
    a1 = _acos(cos1)

    d2 = -(w1x * v1x + w1y * v1y + w1z * v1z)
    n3 = jnp.sqrt(w1x * w1x + w1y * w1y + w1z * w1z)
    cos2 = jnp.clip(d2 / jnp.clip(n3 * bond, 1e-8, None),
                    -0.9999999, 0.9999999)
    a2 = _acos(cos2)

    seq_sel = jnp.where(todo, sc, 0.0).astype(jnp.int32)
    row = lax.broadcasted_iota(jnp.int32, (32, NR), 0)
    oh = jnp.where(row == seq_sel, 1.0, 0.0)
    gathered = jnp.dot(tab_ref[...], oh,
                       preferred_element_type=jnp.float32)

    total = jnp.zeros((1, NR), jnp.float32)
    xs = (bond, a1, a2)
    for i in range(3):
        m = gathered[i:i + 1]
        iv = gathered[3 + i:4 + i]
        tcap = gathered[6 + i:7 + i]
        d = xs[i] - m
        total = total + jnp.minimum(d * d * iv, tcap)

    out_ref[0, 0] = jnp.where(todo, total, 0.0)


def _geom(gf, tab):
    return pl.pallas_call(
        _geom_body,
        grid=(2, BC),
        in_specs=[
            pl.BlockSpec((1, 1, 10, NR), lambda a, b: (a, b, 0, 0)),
            pl.BlockSpec((9, 32), lambda a, b: (0, 0)),
        ],
        out_specs=pl.BlockSpec((1, 1, 1, NR), lambda a, b: (a, b, 0, 0)),
        out_shape=jax.ShapeDtypeStruct((2, BC, 1, NR), jnp.float32),
    )(gf, tab)


def kernel(atom_description, coords, alternatives, mean, std, weight):
    shp = (NA // NR, NR)
    rn = atom_description[:, 0].reshape(shp)
    an = atom_description[:, 1].reshape(shp)
    chn = atom_description[:, 2].reshape(shp)
    resname = atom_description[:, 3]
    bt = atom_description[:, 4].reshape(shp)
    a0 = alternatives[:, 0].astype(jnp.int32).reshape(shp)
    a1 = alternatives[:, 1].astype(jnp.int32).reshape(shp)

    keys = _pack_keys(rn, an, chn, bt, a0, a1).reshape(NA)
    ct = coords.astype(jnp.float32).T
    gf = _sc_build(keys, ct[0], ct[1], ct[2], resname.astype(jnp.int32))

    factor = 1.0 - jnp.tanh(-weight[0])
    var = std.astype(jnp.float32) ** 2
    iv = factor / (2.0 * var)
    tcap = factor * (-jnp.log(jnp.float32(EPS))
                     - jnp.log(jnp.sqrt(2.0 * jnp.pi * var)))
    tab = jnp.zeros((9, 32), jnp.float32)
    tab = tab.at[0:3, :20].set(mean.astype(jnp.float32).T)
    tab = tab.at[3:6, :20].set(iv.T)
    tab = tab.at[6:9, :20].set(tcap.T)

    out = _geom(gf, tab)
    return out.reshape(2, NB, NC, NR).transpose(1, 2, 3, 0)
